# baseline (speedup 1.0000x reference)
import jax
import jax.numpy as jnp
from jax import lax
from jax.experimental import pallas as pl


def _pair_pool_matrix(n, scale):
    r = lax.broadcasted_iota(jnp.int32, (n, n // 2), 0)
    c = lax.broadcasted_iota(jnp.int32, (n, n // 2), 1)
    return jnp.where(r // 2 == c, jnp.float32(scale), jnp.float32(0.0))


def _pool2(x):
    H, W = x.shape
    rows = jnp.sum(x.reshape(H // 2, 2, W), axis=1)
    return jnp.dot(rows, _pair_pool_matrix(W, 0.25),
                   preferred_element_type=jnp.float32)


def _grid_tokens(canvas):
    cols = [canvas[:, 16 * gc:16 * (gc + 1)] for gc in range(8)]
    g = jnp.stack(cols, axis=0)
    g = g.reshape(8, 8, 16, 16).transpose(1, 0, 2, 3)
    return g.reshape(64, 16, 16)


def _ring_tokens(canvas):
    g = _grid_tokens(canvas)
    mids = [g[i:i + 2] for i in (16, 22, 24, 30, 32, 38, 40, 46)]
    return jnp.concatenate([g[0:16]] + mids + [g[48:64]], axis=0)


def _body(x_ref, o_ref):
    x = x_ref[0, 0]
    f2 = _pool2(x)
    f4 = _pool2(f2)
    lvl0 = _grid_tokens(x[192:320, 192:320])
    lvl1 = _ring_tokens(f2[64:192, 64:192])
    lvl2 = _ring_tokens(f4)
    o_ref[0, :, 0] = jnp.concatenate([lvl0, lvl1, lvl2], axis=0)


def kernel(images):
    B, C, H, W = images.shape
    return pl.pallas_call(
        _body,
        grid=(B, C),
        in_specs=[pl.BlockSpec((1, 1, H, W), lambda b, c: (b, c, 0, 0))],
        out_specs=pl.BlockSpec((1, 160, 1, 16, 16),
                               lambda b, c: (b, 0, c, 0, 0)),
        out_shape=jax.ShapeDtypeStruct((B, 160, C, 16, 16), jnp.float32),
    )(images)
